# baseline (speedup 1.0000x reference)
import functools

import jax
import jax.numpy as jnp
from jax import lax
from jax.experimental import pallas as pl
from jax.experimental.pallas import tpu as pltpu
from jax.experimental.pallas import tpu_sc as plsc

_N = 100000
_E = 1600000
_EH = _E // 2
_H = 32
_CHUNK = 2000
_EPW = _EH // 16
_EPT = _E // 16
_BN = 2000
_CHUNK3 = 1000


def _sc_mesh1():
    return plsc.VectorSubcoreMesh(
        core_axis_name="c", subcore_axis_name="s", num_cores=1)


_SC_PARAMS = pltpu.CompilerParams(use_tc_tiling_on_sc=False)


def _deg_call(dst_half, ones_c, zeros_n):
    @functools.partial(
        pl.kernel,
        out_type=jax.ShapeDtypeStruct((_N,), jnp.float32),
        mesh=_sc_mesh1(),
        compiler_params=_SC_PARAMS,
        scratch_types=[
            pltpu.VMEM((_CHUNK,), jnp.int32),
            pltpu.VMEM((_CHUNK,), jnp.float32),
            pltpu.VMEM_SHARED((_N,), jnp.float32),
        ],
    )
    def deg_k(dst_hbm, ones_hbm, zeros_hbm, out_hbm, idx_v, ones_v, acc_sh):
        s = lax.axis_index("s")
        pltpu.sync_copy(ones_hbm, ones_v)

        @pl.when(s == 0)
        def _():
            pltpu.sync_copy(zeros_hbm, acc_sh)

        plsc.subcore_barrier()
        base = s * _EPW

        def body(i, carry):
            off = base + i * _CHUNK
            pltpu.sync_copy(dst_hbm.at[pl.ds(off, _CHUNK)], idx_v)
            pltpu.sync_copy(ones_v, acc_sh.at[idx_v], add=True)
            return carry

        lax.fori_loop(0, _EPW // _CHUNK, body, 0)
        plsc.subcore_barrier()

        @pl.when(s == 0)
        def _():
            pltpu.sync_copy(acc_sh, out_hbm)

    return deg_k(dst_half, ones_c, zeros_n)


def _s1_call(src_half, dst_half, u, zeros_n):
    @functools.partial(
        pl.kernel,
        out_type=jax.ShapeDtypeStruct((_N,), jnp.float32),
        mesh=_sc_mesh1(),
        compiler_params=_SC_PARAMS,
        scratch_types=[
            pltpu.VMEM((_CHUNK,), jnp.int32),
            pltpu.VMEM((_CHUNK,), jnp.int32),
            pltpu.VMEM((_CHUNK,), jnp.float32),
            pltpu.VMEM_SHARED((_N,), jnp.float32),
            pltpu.SemaphoreType.DMA,
        ],
    )
    def s1_k(src_hbm, dst_hbm, u_hbm, zeros_hbm, out_hbm,
             idx_s, idx_d, vals_v, acc_sh, sem):
        s = lax.axis_index("s")

        @pl.when(s == 0)
        def _():
            pltpu.sync_copy(zeros_hbm, acc_sh)

        plsc.subcore_barrier()
        base = s * _EPW

        def body(i, carry):
            off = base + i * _CHUNK
            pltpu.sync_copy(src_hbm.at[pl.ds(off, _CHUNK)], idx_s)
            pltpu.sync_copy(dst_hbm.at[pl.ds(off, _CHUNK)], idx_d)
            pltpu.async_copy(u_hbm.at[idx_s], vals_v, sem).wait()
            pltpu.sync_copy(vals_v, acc_sh.at[idx_d], add=True)
            return carry

        lax.fori_loop(0, _EPW // _CHUNK, body, 0)
        plsc.subcore_barrier()

        @pl.when(s == 0)
        def _():
            pltpu.sync_copy(acc_sh, out_hbm)

    return s1_k(src_half, dst_half, u, zeros_n)


def _s2_call(src, dst, u2x, zeros_n16):
    @functools.partial(
        pl.kernel,
        out_type=jax.ShapeDtypeStruct((_N, 16), jnp.float32),
        mesh=_sc_mesh1(),
        compiler_params=_SC_PARAMS,
        scratch_types=[
            pltpu.VMEM((_CHUNK3,), jnp.int32),
            pltpu.VMEM((_CHUNK3,), jnp.int32),
            pltpu.VMEM((_CHUNK3, 16), jnp.float32),
            pltpu.VMEM_SHARED((_N, 16), jnp.float32),
            pltpu.SemaphoreType.DMA,
        ],
    )
    def s2_k(src_hbm, dst_hbm, u2_hbm, zeros_hbm, out_hbm,
             idx_s, idx_d, rows_v, acc_sh, sem):
        s = lax.axis_index("s")

        @pl.when(s == 0)
        def _():
            pltpu.sync_copy(zeros_hbm, acc_sh)

        plsc.subcore_barrier()
        base = s * _EPT

        def body(i, carry):
            off = base + i * _CHUNK3
            pltpu.sync_copy(src_hbm.at[pl.ds(off, _CHUNK3)], idx_s)
            pltpu.sync_copy(dst_hbm.at[pl.ds(off, _CHUNK3)], idx_d)
            pltpu.async_copy(u2_hbm.at[idx_s], rows_v, sem).wait()
            pltpu.sync_copy(rows_v, acc_sh.at[idx_d], add=True)
            return carry

        lax.fori_loop(0, _EPT // _CHUNK3, body, 0)
        plsc.subcore_barrier()

        @pl.when(s == 0)
        def _():
            pltpu.sync_copy(acc_sh, out_hbm)

    return s2_k(src, dst, u2x, zeros_n16)


def _tc_a(dega, degb, x0):
    def a_k(da_ref, db_ref, x_ref, dinv_ref, u_ref):
        deg = da_ref[:, :] + db_ref[:, :] + 1.0
        dinv = lax.rsqrt(deg)
        dinv_ref[:, :] = dinv
        u_ref[:, :] = dinv * x_ref[:, :]

    return pl.pallas_call(
        a_k,
        out_shape=[
            jax.ShapeDtypeStruct((800, 125), jnp.float32),
            jax.ShapeDtypeStruct((800, 125), jnp.float32),
        ],
    )(dega.reshape(800, 125), degb.reshape(800, 125), x0.reshape(800, 125))


def _tc_b(dinv, s1a, s1b, u, az1, cz1, ah1, ch1):
    def b_k(dinv_ref, s1a_ref, s1b_ref, u_ref, az_ref, czr, ah_ref, chr_,
            u2a_ref, u2b_ref, d16_ref):
        dinv = dinv_ref[:, :]
        y1 = dinv * (s1a_ref[:, :] + s1b_ref[:, :] + u_ref[:, :])
        pz = y1 * az_ref[:, :] + czr[:, :]
        ph = y1 * ah_ref[:, :] + chr_[:, :]
        h1 = jnp.maximum((1.0 - jax.nn.sigmoid(pz)) * jnp.tanh(ph), 0.0)
        u2 = dinv * h1
        u2a_ref[:, :] = u2[:, :16]
        u2b_ref[:, :] = u2[:, 16:]
        d16_ref[:, :] = jnp.broadcast_to(dinv, (_BN, 16))

    grid = _N // _BN
    col = pl.BlockSpec((_BN, 1), lambda i: (i, 0))
    wrow = pl.BlockSpec((1, _H), lambda i: (0, 0))
    half = pl.BlockSpec((_BN, 16), lambda i: (i, 0))
    return pl.pallas_call(
        b_k,
        grid=(grid,),
        in_specs=[col, col, col, col, wrow, wrow, wrow, wrow],
        out_specs=[half, half, half],
        out_shape=[
            jax.ShapeDtypeStruct((_N, 16), jnp.float32),
            jax.ShapeDtypeStruct((_N, 16), jnp.float32),
            jax.ShapeDtypeStruct((_N, 16), jnp.float32),
        ],
    )(dinv.reshape(_N, 1), s1a.reshape(_N, 1), s1b.reshape(_N, 1),
      u.reshape(_N, 1), az1, cz1, ah1, ch1)


def _tc_c(s2a, s2b, u2a, u2b, d16, Az2, cz2, Ah2, ch2, wo, bo):
    def c_k(s2a_ref, s2b_ref, u2a_ref, u2b_ref, d16_ref,
            az_ref, czr, ah_ref, chr_, wo_ref, bo_ref, out_ref):
        d16 = d16_ref[:, :]
        ya = d16 * (s2a_ref[:, :] + u2a_ref[:, :])
        yb = d16 * (s2b_ref[:, :] + u2b_ref[:, :])
        y2 = jnp.concatenate([ya, yb], axis=1)
        pz = jnp.dot(y2, az_ref[:, :], preferred_element_type=jnp.float32)
        ph = jnp.dot(y2, ah_ref[:, :], preferred_element_type=jnp.float32)
        gz = jax.nn.sigmoid(pz + czr[:, :])
        gh = jnp.tanh(ph + chr_[:, :])
        h2 = jnp.maximum((1.0 - gz) * gh, 0.0)
        out_ref[:, :] = (
            jnp.dot(h2, wo_ref[:, :], preferred_element_type=jnp.float32)
            + bo_ref[:, :])

    grid = _N // _BN
    half = pl.BlockSpec((_BN, 16), lambda i: (i, 0))
    wfull = pl.BlockSpec((_H, _H), lambda i: (0, 0))
    wrow = pl.BlockSpec((1, _H), lambda i: (0, 0))
    wcol = pl.BlockSpec((_H, 1), lambda i: (0, 0))
    wone = pl.BlockSpec((1, 1), lambda i: (0, 0))
    col = pl.BlockSpec((_BN, 1), lambda i: (i, 0))
    return pl.pallas_call(
        c_k,
        grid=(grid,),
        in_specs=[half, half, half, half, half,
                  wfull, wrow, wfull, wrow, wcol, wone],
        out_specs=col,
        out_shape=jax.ShapeDtypeStruct((_N, 1), jnp.float32),
    )(s2a, s2b, u2a, u2b, d16, Az2, cz2, Ah2, ch2, wo, bo)


def kernel(x, edge_index, params):
    p = params
    src = edge_index[0]
    dst = edge_index[1]
    x0 = x[0, :, 0, 0]

    az1 = p['Wc_z1'] @ p['Wl_z1'][:_H]
    cz1 = (p['bc_z1'] @ p['Wl_z1'][:_H] + p['bl_z1'])[None]
    ah1 = p['Wc_h1'] @ p['Wl_h1'][:_H]
    ch1 = (p['bc_h1'] @ p['Wl_h1'][:_H] + p['bl_h1'])[None]
    Az2 = p['Wc_z2'] @ p['Wl_z2'][:_H]
    cz2 = (p['bc_z2'] @ p['Wl_z2'][:_H] + p['bl_z2'])[None]
    Ah2 = p['Wc_h2'] @ p['Wl_h2'][:_H]
    ch2 = (p['bc_h2'] @ p['Wl_h2'][:_H] + p['bl_h2'])[None]
    wo = p['W_out']
    bo = p['b_out'][None]

    ones_c = jnp.ones((_CHUNK,), jnp.float32)
    zeros_n = jnp.zeros((_N,), jnp.float32)
    zeros_n16 = jnp.zeros((_N, 16), jnp.float32)

    dega = _deg_call(dst[:_EH], ones_c, zeros_n)
    degb = _deg_call(dst[_EH:], ones_c, zeros_n)
    dinv, u = _tc_a(dega, degb, x0)
    dinv = dinv.reshape(_N)
    u = u.reshape(_N)
    s1a = _s1_call(src[:_EH], dst[:_EH], u, zeros_n)
    s1b = _s1_call(src[_EH:], dst[_EH:], u, zeros_n)
    u2a, u2b, d16 = _tc_b(dinv, s1a, s1b, u, az1, cz1, ah1, ch1)
    s2a = _s2_call(src, dst, u2a, zeros_n16)
    s2b = _s2_call(src, dst, u2b, zeros_n16)
    out = _tc_c(s2a, s2b, u2a, u2b, d16, Az2, cz2, Ah2, ch2, wo, bo)
    return out.reshape(1, _N, 1)


# speedup vs baseline: 1.5981x; 1.5981x over previous
import functools

import jax
import jax.numpy as jnp
from jax import lax
from jax.experimental import pallas as pl
from jax.experimental.pallas import tpu as pltpu
from jax.experimental.pallas import tpu_sc as plsc

_N = 100000
_E = 1600000
_H = 32
_NW = 32
_EPW = _E // _NW
_EPT = _E // 16
_C12 = 2000
_C3 = 400
_BN = 2000


def _sc_mesh():
    return plsc.VectorSubcoreMesh(core_axis_name="c", subcore_axis_name="s")


_SC_PARAMS = pltpu.CompilerParams(use_tc_tiling_on_sc=False)

_DMA8 = [pltpu.SemaphoreType.DMA] * 8


def _pipeline(nit, idx_cp, one_step):
    idx_cp(0, 0).start()
    idx_cp(1, 1).start()
    quads = nit // 4

    def body4(io, carry):
        for k in range(4):
            one_step(io * 4 + k, k, True)
        return carry

    if quads:
        lax.fori_loop(0, quads, body4, 0)
    for k in range(nit % 4):
        one_step(quads * 4 + k, k, False)


def _deg_call(dst, ones_c, zeros_n):
    @functools.partial(
        pl.kernel,
        out_type=jax.ShapeDtypeStruct((2, _N), jnp.float32),
        mesh=_sc_mesh(),
        compiler_params=_SC_PARAMS,
        scratch_types=[
            pltpu.VMEM((4, _C12), jnp.int32),
            pltpu.VMEM((_C12,), jnp.float32),
            pltpu.VMEM_SHARED((_N,), jnp.float32),
        ] + _DMA8[:6],
    )
    def deg_k(dst_hbm, ones_hbm, zeros_hbm, out_hbm, idx_v, ones_v, acc_sh,
              is0, is1, is2, is3, ss0, ss1):
        c = lax.axis_index("c")
        s = lax.axis_index("s")
        wid = s * 2 + c
        isem = (is0, is1, is2, is3)
        ssem = (ss0, ss1)
        pltpu.sync_copy(ones_hbm, ones_v)

        @pl.when(s == 0)
        def _():
            pltpu.sync_copy(zeros_hbm, acc_sh)

        plsc.subcore_barrier()
        base = wid * _EPW
        nit = _EPW // _C12

        def idx_cp(i, q):
            return pltpu.make_async_copy(
                dst_hbm.at[pl.ds(base + i * _C12, _C12)], idx_v.at[q],
                isem[q])

        def sc_wait(q, r):
            pltpu.make_async_copy(
                ones_v, acc_sh.at[idx_v.at[q]], ssem[r]).wait()

        def one_step(i, k, dyn):
            q, r = k, k % 2
            idx_cp(i, q).wait()
            if dyn:
                @pl.when(i >= 2)
                def _():
                    sc_wait((k + 2) % 4, r)

                @pl.when(i + 2 < nit)
                def _():
                    idx_cp(i + 2, (k + 2) % 4).start()
            else:
                if i >= 2:
                    sc_wait((k + 2) % 4, r)
                if i + 2 < nit:
                    idx_cp(i + 2, (k + 2) % 4).start()
            pltpu.async_copy(
                ones_v, acc_sh.at[idx_v.at[q]], ssem[r], add=True)

        _pipeline(nit, idx_cp, one_step)
        sc_wait(0, (nit - 2) % 2)
        sc_wait(1, (nit - 1) % 2)
        plsc.subcore_barrier()

        @pl.when(s == 0)
        def _():
            pltpu.sync_copy(acc_sh, out_hbm.at[c])

    return deg_k(dst, ones_c, zeros_n)


def _s1_call(edge_index, u, zeros_n):
    @functools.partial(
        pl.kernel,
        out_type=jax.ShapeDtypeStruct((2, _N), jnp.float32),
        mesh=_sc_mesh(),
        compiler_params=_SC_PARAMS,
        scratch_types=[
            pltpu.VMEM((4, 2, _C12), jnp.int32),
            pltpu.VMEM((2, _C12), jnp.float32),
            pltpu.VMEM_SHARED((_N,), jnp.float32),
            pltpu.VMEM_SHARED((_N,), jnp.float32),
        ] + _DMA8,
    )
    def s1_k(ei_hbm, u_hbm, zeros_hbm, out_hbm, idx_v, vals_v, u_sh, acc_sh,
             is0, is1, is2, is3, ss0, ss1, gs0, gs1):
        c = lax.axis_index("c")
        s = lax.axis_index("s")
        wid = s * 2 + c
        isem = (is0, is1, is2, is3)
        ssem = (ss0, ss1)
        gsem = (gs0, gs1)

        @pl.when(s == 0)
        def _():
            pltpu.sync_copy(zeros_hbm, acc_sh)
            pltpu.sync_copy(u_hbm, u_sh)

        plsc.subcore_barrier()
        base = wid * _EPW
        nit = _EPW // _C12

        def idx_cp(i, q):
            return pltpu.make_async_copy(
                ei_hbm.at[:, pl.ds(base + i * _C12, _C12)], idx_v.at[q],
                isem[q])

        def sc_wait(q, r):
            pltpu.make_async_copy(
                vals_v.at[r], acc_sh.at[idx_v.at[q, 1]], ssem[r]).wait()

        def one_step(i, k, dyn):
            q, r = k, k % 2
            idx_cp(i, q).wait()
            if dyn:
                @pl.when(i >= 2)
                def _():
                    sc_wait((k + 2) % 4, r)

                @pl.when(i + 2 < nit)
                def _():
                    idx_cp(i + 2, (k + 2) % 4).start()
            else:
                if i >= 2:
                    sc_wait((k + 2) % 4, r)
                if i + 2 < nit:
                    idx_cp(i + 2, (k + 2) % 4).start()
            pltpu.async_copy(
                u_sh.at[idx_v.at[q, 0]], vals_v.at[r], gsem[r]).wait()
            pltpu.async_copy(
                vals_v.at[r], acc_sh.at[idx_v.at[q, 1]], ssem[r], add=True)

        _pipeline(nit, idx_cp, one_step)
        sc_wait(0, (nit - 2) % 2)
        sc_wait(1, (nit - 1) % 2)
        plsc.subcore_barrier()

        @pl.when(s == 0)
        def _():
            pltpu.sync_copy(acc_sh, out_hbm.at[c])

    return s1_k(edge_index, u, zeros_n)


def _s2_call(edge_index, u2a, u2b, zeros_n16):
    @functools.partial(
        pl.kernel,
        out_type=jax.ShapeDtypeStruct((2, _N, 16), jnp.float32),
        mesh=_sc_mesh(),
        compiler_params=_SC_PARAMS,
        scratch_types=[
            pltpu.VMEM((4, 2, _C3), jnp.int32),
            pltpu.VMEM((2, _C3, 16), jnp.float32),
            pltpu.VMEM_SHARED((_N, 16), jnp.float32),
        ] + _DMA8,
    )
    def s2_k(ei_hbm, u2a_hbm, u2b_hbm, zeros_hbm, out_hbm, idx_v, rows_v,
             acc_sh, is0, is1, is2, is3, ss0, ss1, gs0, gs1):
        c = lax.axis_index("c")
        s = lax.axis_index("s")
        isem = (is0, is1, is2, is3)
        ssem = (ss0, ss1)
        gsem = (gs0, gs1)

        @pl.when(s == 0)
        def _():
            pltpu.sync_copy(zeros_hbm, acc_sh)

        plsc.subcore_barrier()
        base = s * _EPT
        nit = _EPT // _C3

        def idx_cp(i, q):
            return pltpu.make_async_copy(
                ei_hbm.at[:, pl.ds(base + i * _C3, _C3)], idx_v.at[q],
                isem[q])

        def sc_wait(q, r):
            pltpu.make_async_copy(
                rows_v.at[r], acc_sh.at[idx_v.at[q, 1]], ssem[r]).wait()

        def one_step(i, k, dyn):
            q, r = k, k % 2
            idx_cp(i, q).wait()
            if dyn:
                @pl.when(i >= 2)
                def _():
                    sc_wait((k + 2) % 4, r)

                @pl.when(i + 2 < nit)
                def _():
                    idx_cp(i + 2, (k + 2) % 4).start()
            else:
                if i >= 2:
                    sc_wait((k + 2) % 4, r)
                if i + 2 < nit:
                    idx_cp(i + 2, (k + 2) % 4).start()

            @pl.when(c == 0)
            def _():
                pltpu.async_copy(
                    u2a_hbm.at[idx_v.at[q, 0]], rows_v.at[r],
                    gsem[r]).wait()

            @pl.when(c == 1)
            def _():
                pltpu.async_copy(
                    u2b_hbm.at[idx_v.at[q, 0]], rows_v.at[r],
                    gsem[r]).wait()

            pltpu.async_copy(
                rows_v.at[r], acc_sh.at[idx_v.at[q, 1]], ssem[r], add=True)

        _pipeline(nit, idx_cp, one_step)
        sc_wait(0, (nit - 2) % 2)
        sc_wait(1, (nit - 1) % 2)
        plsc.subcore_barrier()

        @pl.when(s == 0)
        def _():
            pltpu.sync_copy(acc_sh, out_hbm.at[c])

    return s2_k(edge_index, u2a, u2b, zeros_n16)


def _tc_a(deg_parts, x0):
    def a_k(parts_ref, x_ref, dinv_ref, u_ref):
        deg = parts_ref[0] + parts_ref[1] + 1.0
        dinv = lax.rsqrt(deg)
        dinv_ref[:, :] = dinv
        u_ref[:, :] = dinv * x_ref[:, :]

    return pl.pallas_call(
        a_k,
        out_shape=[
            jax.ShapeDtypeStruct((800, 125), jnp.float32),
            jax.ShapeDtypeStruct((800, 125), jnp.float32),
        ],
    )(deg_parts.reshape(2, 800, 125), x0.reshape(800, 125))


def _tc_b(dinv, s1a, s1b, u, az1, cz1, ah1, ch1):
    def b_k(dinv_ref, s1a_ref, s1b_ref, u_ref, az_ref, czr, ah_ref, chr_,
            u2a_ref, u2b_ref, d16_ref):
        dinv = dinv_ref[:, :]
        y1 = dinv * (s1a_ref[:, :] + s1b_ref[:, :] + u_ref[:, :])
        pz = y1 * az_ref[:, :] + czr[:, :]
        ph = y1 * ah_ref[:, :] + chr_[:, :]
        h1 = jnp.maximum((1.0 - jax.nn.sigmoid(pz)) * jnp.tanh(ph), 0.0)
        u2 = dinv * h1
        u2a_ref[:, :] = u2[:, :16]
        u2b_ref[:, :] = u2[:, 16:]
        d16_ref[:, :] = jnp.broadcast_to(dinv, (_BN, 16))

    grid = _N // _BN
    col = pl.BlockSpec((_BN, 1), lambda i: (i, 0))
    wrow = pl.BlockSpec((1, _H), lambda i: (0, 0))
    half = pl.BlockSpec((_BN, 16), lambda i: (i, 0))
    return pl.pallas_call(
        b_k,
        grid=(grid,),
        in_specs=[col, col, col, col, wrow, wrow, wrow, wrow],
        out_specs=[half, half, half],
        out_shape=[
            jax.ShapeDtypeStruct((_N, 16), jnp.float32),
            jax.ShapeDtypeStruct((_N, 16), jnp.float32),
            jax.ShapeDtypeStruct((_N, 16), jnp.float32),
        ],
    )(dinv.reshape(_N, 1), s1a.reshape(_N, 1), s1b.reshape(_N, 1),
      u.reshape(_N, 1), az1, cz1, ah1, ch1)


def _tc_c(s2a, s2b, u2a, u2b, d16, Az2, cz2, Ah2, ch2, wo, bo):
    def c_k(s2a_ref, s2b_ref, u2a_ref, u2b_ref, d16_ref,
            az_ref, czr, ah_ref, chr_, wo_ref, bo_ref, out_ref):
        d16 = d16_ref[:, :]
        ya = d16 * (s2a_ref[:, :] + u2a_ref[:, :])
        yb = d16 * (s2b_ref[:, :] + u2b_ref[:, :])
        y2 = jnp.concatenate([ya, yb], axis=1)
        pz = jnp.dot(y2, az_ref[:, :], preferred_element_type=jnp.float32)
        ph = jnp.dot(y2, ah_ref[:, :], preferred_element_type=jnp.float32)
        gz = jax.nn.sigmoid(pz + czr[:, :])
        gh = jnp.tanh(ph + chr_[:, :])
        h2 = jnp.maximum((1.0 - gz) * gh, 0.0)
        out_ref[:, :] = (
            jnp.dot(h2, wo_ref[:, :], preferred_element_type=jnp.float32)
            + bo_ref[:, :])

    grid = _N // _BN
    half = pl.BlockSpec((_BN, 16), lambda i: (i, 0))
    wfull = pl.BlockSpec((_H, _H), lambda i: (0, 0))
    wrow = pl.BlockSpec((1, _H), lambda i: (0, 0))
    wcol = pl.BlockSpec((_H, 1), lambda i: (0, 0))
    wone = pl.BlockSpec((1, 1), lambda i: (0, 0))
    col = pl.BlockSpec((_BN, 1), lambda i: (i, 0))
    return pl.pallas_call(
        c_k,
        grid=(grid,),
        in_specs=[half, half, half, half, half,
                  wfull, wrow, wfull, wrow, wcol, wone],
        out_specs=col,
        out_shape=jax.ShapeDtypeStruct((_N, 1), jnp.float32),
    )(s2a, s2b, u2a, u2b, d16, Az2, cz2, Ah2, ch2, wo, bo)


def kernel(x, edge_index, params):
    p = params
    dst = edge_index[1]
    x0 = x[0, :, 0, 0]

    az1 = p['Wc_z1'] @ p['Wl_z1'][:_H]
    cz1 = (p['bc_z1'] @ p['Wl_z1'][:_H] + p['bl_z1'])[None]
    ah1 = p['Wc_h1'] @ p['Wl_h1'][:_H]
    ch1 = (p['bc_h1'] @ p['Wl_h1'][:_H] + p['bl_h1'])[None]
    Az2 = p['Wc_z2'] @ p['Wl_z2'][:_H]
    cz2 = (p['bc_z2'] @ p['Wl_z2'][:_H] + p['bl_z2'])[None]
    Ah2 = p['Wc_h2'] @ p['Wl_h2'][:_H]
    ch2 = (p['bc_h2'] @ p['Wl_h2'][:_H] + p['bl_h2'])[None]
    wo = p['W_out']
    bo = p['b_out'][None]

    ones_c = jnp.ones((_C12,), jnp.float32)
    zeros_n = jnp.zeros((_N,), jnp.float32)
    zeros_n16 = jnp.zeros((_N, 16), jnp.float32)

    deg_parts = _deg_call(dst, ones_c, zeros_n)
    dinv, u = _tc_a(deg_parts, x0)
    dinv = dinv.reshape(_N)
    u = u.reshape(_N)
    s1 = _s1_call(edge_index, u, zeros_n)
    u2a, u2b, d16 = _tc_b(dinv, s1[0], s1[1], u, az1, cz1, ah1, ch1)
    s2 = _s2_call(edge_index, u2a, u2b, zeros_n16)
    out = _tc_c(s2[0], s2[1], u2a, u2b, d16, Az2, cz2, Ah2, ch2, wo, bo)
    return out.reshape(1, _N, 1)


# speedup vs baseline: 2.0838x; 1.3039x over previous
import functools

import jax
import jax.numpy as jnp
from jax import lax
from jax.experimental import pallas as pl
from jax.experimental.pallas import tpu as pltpu
from jax.experimental.pallas import tpu_sc as plsc

_N = 100000
_E = 1600000
_H = 32
_NW = 32
_EPW = _E // _NW
_EPT = _E // 16
_C12 = 2000
_C3 = 400
_BN = 2000


def _sc_mesh():
    return plsc.VectorSubcoreMesh(core_axis_name="c", subcore_axis_name="s")


_SC_PARAMS = pltpu.CompilerParams(use_tc_tiling_on_sc=False)

_DMA8 = [pltpu.SemaphoreType.DMA] * 8


def _pipeline(nit, idx_cp, one_step):
    idx_cp(0, 0).start()
    idx_cp(1, 1).start()
    quads = nit // 4

    def body4(io, carry):
        for k in range(4):
            one_step(io * 4 + k, k, True)
        return carry

    if quads:
        lax.fori_loop(0, quads, body4, 0)
    for k in range(nit % 4):
        one_step(quads * 4 + k, k, False)


def _deg_call(dst, ones_c, zeros_n):
    @functools.partial(
        pl.kernel,
        out_type=jax.ShapeDtypeStruct((2, _N), jnp.float32),
        mesh=_sc_mesh(),
        compiler_params=_SC_PARAMS,
        scratch_types=[
            pltpu.VMEM((4, _C12), jnp.int32),
            pltpu.VMEM((_C12,), jnp.float32),
            pltpu.VMEM_SHARED((_N,), jnp.float32),
        ] + _DMA8[:6],
    )
    def deg_k(dst_hbm, ones_hbm, zeros_hbm, out_hbm, idx_v, ones_v, acc_sh,
              is0, is1, is2, is3, ss0, ss1):
        c = lax.axis_index("c")
        s = lax.axis_index("s")
        wid = s * 2 + c
        isem = (is0, is1, is2, is3)
        ssem = (ss0, ss1)
        pltpu.sync_copy(ones_hbm, ones_v)

        @pl.when(s == 0)
        def _():
            pltpu.sync_copy(zeros_hbm, acc_sh)

        plsc.subcore_barrier()
        base = wid * _EPW
        nit = _EPW // _C12

        def idx_cp(i, q):
            return pltpu.make_async_copy(
                dst_hbm.at[pl.ds(base + i * _C12, _C12)], idx_v.at[q],
                isem[q])

        def sc_wait(q, r):
            pltpu.make_async_copy(
                ones_v, acc_sh.at[idx_v.at[q]], ssem[r]).wait()

        def one_step(i, k, dyn):
            q, r = k, k % 2
            idx_cp(i, q).wait()
            if dyn:
                @pl.when(i >= 2)
                def _():
                    sc_wait((k + 2) % 4, r)

                @pl.when(i + 2 < nit)
                def _():
                    idx_cp(i + 2, (k + 2) % 4).start()
            else:
                if i >= 2:
                    sc_wait((k + 2) % 4, r)
                if i + 2 < nit:
                    idx_cp(i + 2, (k + 2) % 4).start()
            pltpu.async_copy(
                ones_v, acc_sh.at[idx_v.at[q]], ssem[r], add=True)

        _pipeline(nit, idx_cp, one_step)
        sc_wait(0, (nit - 2) % 2)
        sc_wait(1, (nit - 1) % 2)
        plsc.subcore_barrier()

        @pl.when(s == 0)
        def _():
            pltpu.sync_copy(acc_sh, out_hbm.at[c])

    return deg_k(dst, ones_c, zeros_n)


def _s1_call(edge_index, u, zeros_n):
    @functools.partial(
        pl.kernel,
        out_type=jax.ShapeDtypeStruct((2, _N), jnp.float32),
        mesh=_sc_mesh(),
        compiler_params=_SC_PARAMS,
        scratch_types=[
            pltpu.VMEM((4, 2, _C12), jnp.int32),
            pltpu.VMEM((2, _C12), jnp.float32),
            pltpu.VMEM_SHARED((_N,), jnp.float32),
            pltpu.VMEM_SHARED((_N,), jnp.float32),
        ] + _DMA8,
    )
    def s1_k(ei_hbm, u_hbm, zeros_hbm, out_hbm, idx_v, vals_v, u_sh, acc_sh,
             is0, is1, is2, is3, ss0, ss1, gs0, gs1):
        c = lax.axis_index("c")
        s = lax.axis_index("s")
        wid = s * 2 + c
        isem = (is0, is1, is2, is3)
        ssem = (ss0, ss1)
        gsem = (gs0, gs1)

        @pl.when(s == 0)
        def _():
            pltpu.sync_copy(zeros_hbm, acc_sh)
            pltpu.sync_copy(u_hbm, u_sh)

        plsc.subcore_barrier()
        base = wid * _EPW
        nit = _EPW // _C12

        def idx_cp(i, q):
            return pltpu.make_async_copy(
                ei_hbm.at[:, pl.ds(base + i * _C12, _C12)], idx_v.at[q],
                isem[q])

        def sc_wait(q, r):
            pltpu.make_async_copy(
                vals_v.at[r], acc_sh.at[idx_v.at[q, 1]], ssem[r]).wait()

        def one_step(i, k, dyn):
            q, r = k, k % 2
            idx_cp(i, q).wait()
            if dyn:
                @pl.when(i >= 2)
                def _():
                    sc_wait((k + 2) % 4, r)

                @pl.when(i + 2 < nit)
                def _():
                    idx_cp(i + 2, (k + 2) % 4).start()
            else:
                if i >= 2:
                    sc_wait((k + 2) % 4, r)
                if i + 2 < nit:
                    idx_cp(i + 2, (k + 2) % 4).start()
            pltpu.async_copy(
                u_sh.at[idx_v.at[q, 0]], vals_v.at[r], gsem[r]).wait()
            pltpu.async_copy(
                vals_v.at[r], acc_sh.at[idx_v.at[q, 1]], ssem[r], add=True)

        _pipeline(nit, idx_cp, one_step)
        sc_wait(0, (nit - 2) % 2)
        sc_wait(1, (nit - 1) % 2)
        plsc.subcore_barrier()

        @pl.when(s == 0)
        def _():
            pltpu.sync_copy(acc_sh, out_hbm.at[c])

    return s1_k(edge_index, u, zeros_n)


def _s2_call(edge_index, u2a, u2b):
    @functools.partial(
        pl.kernel,
        out_type=jax.ShapeDtypeStruct((2, _N, 16), jnp.float32),
        mesh=_sc_mesh(),
        compiler_params=_SC_PARAMS,
        scratch_types=[
            pltpu.VMEM((4, 2, _C3), jnp.int32),
            pltpu.VMEM((2, _C3, 16), jnp.float32),
            pltpu.VMEM_SHARED((_N, 16), jnp.float32),
        ] + _DMA8,
    )
    def s2_k(ei_hbm, u2a_hbm, u2b_hbm, out_hbm, idx_v, rows_v,
             acc_sh, is0, is1, is2, is3, ss0, ss1, gs0, gs1):
        c = lax.axis_index("c")
        s = lax.axis_index("s")
        isem = (is0, is1, is2, is3)
        ssem = (ss0, ss1)
        gsem = (gs0, gs1)

        @pl.when((s == 0) & (c == 0))
        def _():
            pltpu.sync_copy(u2a_hbm, acc_sh)

        @pl.when((s == 0) & (c == 1))
        def _():
            pltpu.sync_copy(u2b_hbm, acc_sh)

        plsc.subcore_barrier()
        base = s * _EPT
        nit = _EPT // _C3

        def idx_cp(i, q):
            return pltpu.make_async_copy(
                ei_hbm.at[:, pl.ds(base + i * _C3, _C3)], idx_v.at[q],
                isem[q])

        def sc_wait(q, r):
            pltpu.make_async_copy(
                rows_v.at[r], acc_sh.at[idx_v.at[q, 1]], ssem[r]).wait()

        def one_step(i, k, dyn):
            q, r = k, k % 2
            idx_cp(i, q).wait()
            if dyn:
                @pl.when(i >= 2)
                def _():
                    sc_wait((k + 2) % 4, r)

                @pl.when(i + 2 < nit)
                def _():
                    idx_cp(i + 2, (k + 2) % 4).start()
            else:
                if i >= 2:
                    sc_wait((k + 2) % 4, r)
                if i + 2 < nit:
                    idx_cp(i + 2, (k + 2) % 4).start()

            @pl.when(c == 0)
            def _():
                pltpu.async_copy(
                    u2a_hbm.at[idx_v.at[q, 0]], rows_v.at[r],
                    gsem[r]).wait()

            @pl.when(c == 1)
            def _():
                pltpu.async_copy(
                    u2b_hbm.at[idx_v.at[q, 0]], rows_v.at[r],
                    gsem[r]).wait()

            pltpu.async_copy(
                rows_v.at[r], acc_sh.at[idx_v.at[q, 1]], ssem[r], add=True)

        _pipeline(nit, idx_cp, one_step)
        sc_wait(0, (nit - 2) % 2)
        sc_wait(1, (nit - 1) % 2)
        plsc.subcore_barrier()

        @pl.when(s == 0)
        def _():
            pltpu.sync_copy(acc_sh, out_hbm.at[c])

    return s2_k(edge_index, u2a, u2b)


def _tc_a(deg_parts, x0):
    def a_k(parts_ref, x_ref, dinv_ref, u_ref):
        deg = parts_ref[0] + parts_ref[1] + 1.0
        dinv = lax.rsqrt(deg)
        dinv_ref[:, :] = dinv
        u_ref[:, :] = dinv * x_ref[:, :]

    return pl.pallas_call(
        a_k,
        out_shape=[
            jax.ShapeDtypeStruct((800, 125), jnp.float32),
            jax.ShapeDtypeStruct((800, 125), jnp.float32),
        ],
    )(deg_parts.reshape(2, 800, 125), x0.reshape(800, 125))


def _tc_b(dinv, s1a, s1b, u, az1, cz1, ah1, ch1):
    def b_k(dinv_ref, s1a_ref, s1b_ref, u_ref, az_ref, czr, ah_ref, chr_,
            u2a_ref, u2b_ref):
        dinv = dinv_ref[:, :]
        y1 = dinv * (s1a_ref[:, :] + s1b_ref[:, :] + u_ref[:, :])
        pz = y1 * az_ref[:, :] + czr[:, :]
        ph = y1 * ah_ref[:, :] + chr_[:, :]
        h1 = jnp.maximum((1.0 - jax.nn.sigmoid(pz)) * jnp.tanh(ph), 0.0)
        u2 = dinv * h1
        u2a_ref[:, :] = u2[:, :16]
        u2b_ref[:, :] = u2[:, 16:]

    bn = 5000
    grid = _N // bn
    col = pl.BlockSpec((bn, 1), lambda i: (i, 0))
    wrow = pl.BlockSpec((1, _H), lambda i: (0, 0))
    half = pl.BlockSpec((bn, 16), lambda i: (i, 0))
    return pl.pallas_call(
        b_k,
        grid=(grid,),
        in_specs=[col, col, col, col, wrow, wrow, wrow, wrow],
        out_specs=[half, half],
        out_shape=[
            jax.ShapeDtypeStruct((_N, 16), jnp.float32),
            jax.ShapeDtypeStruct((_N, 16), jnp.float32),
        ],
    )(dinv.reshape(_N, 1), s1a.reshape(_N, 1), s1b.reshape(_N, 1),
      u.reshape(_N, 1), az1, cz1, ah1, ch1)


def _tc_c(s2v, dinv8, r8, kz, kh, ck, kw, bo):
    def c_k(s2_ref, d8_ref, r8_ref, kz_ref, kh_ref, ck_ref, kw_ref, bo_ref,
            out_ref):
        f32 = jnp.float32
        d128 = jnp.dot(d8_ref[:, :], r8_ref[:, :], preferred_element_type=f32)
        ya = d128 * s2_ref[0]
        yb = d128 * s2_ref[1]

        def pre(kref, c_lo, c_hi):
            lo = (jnp.dot(ya, kref[0], preferred_element_type=f32)
                  + jnp.dot(yb, kref[1], preferred_element_type=f32)
                  + c_lo)
            hi = (jnp.dot(ya, kref[2], preferred_element_type=f32)
                  + jnp.dot(yb, kref[3], preferred_element_type=f32)
                  + c_hi)
            return lo, hi

        pz_lo, pz_hi = pre(kz_ref, ck_ref[0:1, :], ck_ref[1:2, :])
        ph_lo, ph_hi = pre(kh_ref, ck_ref[2:3, :], ck_ref[3:4, :])
        h_lo = jnp.maximum(
            (1.0 - jax.nn.sigmoid(pz_lo)) * jnp.tanh(ph_lo), 0.0)
        h_hi = jnp.maximum(
            (1.0 - jax.nn.sigmoid(pz_hi)) * jnp.tanh(ph_hi), 0.0)
        out_ref[:, :] = (
            jnp.dot(h_lo, kw_ref[0], preferred_element_type=f32)
            + jnp.dot(h_hi, kw_ref[1], preferred_element_type=f32)
            + bo_ref[:, :])

    return pl.pallas_call(
        c_k,
        out_shape=jax.ShapeDtypeStruct((_N // 8, 8), jnp.float32),
    )(s2v, dinv8, r8, kz, kh, ck, kw, bo)


def kernel(x, edge_index, params):
    p = params
    dst = edge_index[1]
    x0 = x[0, :, 0, 0]

    az1 = p['Wc_z1'] @ p['Wl_z1'][:_H]
    cz1 = (p['bc_z1'] @ p['Wl_z1'][:_H] + p['bl_z1'])[None]
    ah1 = p['Wc_h1'] @ p['Wl_h1'][:_H]
    ch1 = (p['bc_h1'] @ p['Wl_h1'][:_H] + p['bl_h1'])[None]
    Az2 = p['Wc_z2'] @ p['Wl_z2'][:_H]
    cz2 = (p['bc_z2'] @ p['Wl_z2'][:_H] + p['bl_z2'])[None]
    Ah2 = p['Wc_h2'] @ p['Wl_h2'][:_H]
    ch2 = (p['bc_h2'] @ p['Wl_h2'][:_H] + p['bl_h2'])[None]
    wo = p['W_out']
    bo = p['b_out'][None]

    eye8 = jnp.eye(8, dtype=jnp.float32)
    kz = jnp.stack([
        jnp.kron(eye8, Az2[:16, :16]), jnp.kron(eye8, Az2[16:, :16]),
        jnp.kron(eye8, Az2[:16, 16:]), jnp.kron(eye8, Az2[16:, 16:])])
    kh = jnp.stack([
        jnp.kron(eye8, Ah2[:16, :16]), jnp.kron(eye8, Ah2[16:, :16]),
        jnp.kron(eye8, Ah2[:16, 16:]), jnp.kron(eye8, Ah2[16:, 16:])])
    ck = jnp.stack([
        jnp.tile(cz2[0, :16], 8), jnp.tile(cz2[0, 16:], 8),
        jnp.tile(ch2[0, :16], 8), jnp.tile(ch2[0, 16:], 8)])
    kw = jnp.stack([jnp.kron(eye8, wo[:16]), jnp.kron(eye8, wo[16:])])
    r8 = (jnp.arange(128)[None, :] // 16
          == jnp.arange(8)[:, None]).astype(jnp.float32)

    ones_c = jnp.ones((_C12,), jnp.float32)
    zeros_n = jnp.zeros((_N,), jnp.float32)

    deg_parts = _deg_call(dst, ones_c, zeros_n)
    dinv, u = _tc_a(deg_parts, x0)
    dinv = dinv.reshape(_N)
    u = u.reshape(_N)
    s1 = _s1_call(edge_index, u, zeros_n)
    u2a, u2b = _tc_b(dinv, s1[0], s1[1], u, az1, cz1, ah1, ch1)
    s2 = _s2_call(edge_index, u2a, u2b)
    s2v = s2.reshape(2, _N // 8, 128)
    out = _tc_c(s2v, dinv.reshape(_N // 8, 8), r8, kz, kh, ck, kw, bo)
    return out.reshape(1, _N, 1)
